# Initial kernel scaffold; baseline (speedup 1.0000x reference)
import jax
import jax.numpy as jnp
from jax.experimental import pallas as pl


def kernel(new_node_features, existing_node_features, W_fc, b_fc, W_om, b_om):
    raise NotImplementedError("write your pallas kernel here")


# speedup vs baseline: 2.2180x; 2.2180x over previous
import jax
import jax.numpy as jnp
from jax.experimental import pallas as pl
from jax.experimental.pallas import tpu as pltpu
from jax.experimental.pallas import tpu_sc as plsc

_K_BLOCK = 2000
_GATHER_WINDOW = 128
_NEG = -3.0e38
_HIGHEST = jax.lax.Precision.HIGHEST


def _topk_body(q_ref, k_ref, v1o, v2o, v3o, i1o, i2o, i3o,
               qn, v1, v2, v3, i1, i2, i3):
    step = pl.program_id(0)
    nsteps = pl.num_programs(0)

    @pl.when(step == 0)
    def _():
        q0 = q_ref[...]
        qn[...] = jnp.sqrt(jnp.sum(q0 * q0, axis=1, keepdims=True))
        v1[...] = jnp.full(v1.shape, _NEG, jnp.float32)
        v2[...] = jnp.full(v2.shape, _NEG, jnp.float32)
        v3[...] = jnp.full(v3.shape, _NEG, jnp.float32)
        i1[...] = jnp.zeros(i1.shape, jnp.int32)
        i2[...] = jnp.zeros(i2.shape, jnp.int32)
        i3[...] = jnp.zeros(i3.shape, jnp.int32)

    kb = k_ref[...]
    sq = kb * kb
    ones = jnp.ones((8, kb.shape[1]), jnp.float32)
    knsq = jax.lax.dot_general(ones, sq, (((1,), (1,)), ((), ())),
                               preferred_element_type=jnp.float32,
                               precision=_HIGHEST)[0:1]
    kn = jnp.sqrt(knsq)
    s = jax.lax.dot_general(q_ref[...], kb, (((1,), (1,)), ((), ())),
                            preferred_element_type=jnp.float32)
    sim = s / (qn[...] * kn + 1e-8)

    iota = jax.lax.broadcasted_iota(jnp.int32, sim.shape, 1) + step * _K_BLOCK
    big = jnp.int32(2 ** 30)

    m1 = jnp.max(sim, axis=1, keepdims=True)
    b1 = jnp.min(jnp.where(sim == m1, iota, big), axis=1, keepdims=True)
    sim_m = jnp.where(iota == b1, _NEG, sim)
    m2 = jnp.max(sim_m, axis=1, keepdims=True)
    b2 = jnp.min(jnp.where(sim_m == m2, iota, big), axis=1, keepdims=True)
    sim_m2 = jnp.where(iota == b2, _NEG, sim_m)
    m3 = jnp.max(sim_m2, axis=1, keepdims=True)
    b3 = jnp.min(jnp.where(sim_m2 == m3, iota, big), axis=1, keepdims=True)

    def _insert(cv, ci):
        pv1, pv2, pv3 = v1[...], v2[...], v3[...]
        pi1, pi2, pi3 = i1[...], i2[...], i3[...]
        g1 = cv > pv1
        g2 = cv > pv2
        g3 = cv > pv3
        v1[...] = jnp.where(g1, cv, pv1)
        i1[...] = jnp.where(g1, ci, pi1)
        v2[...] = jnp.where(g1, pv1, jnp.where(g2, cv, pv2))
        i2[...] = jnp.where(g1, pi1, jnp.where(g2, ci, pi2))
        v3[...] = jnp.where(g1 | g2, pv2, jnp.where(g3, cv, pv3))
        i3[...] = jnp.where(g1 | g2, pi2, jnp.where(g3, ci, pi3))

    _insert(m1, b1)
    _insert(m2, b2)
    _insert(m3, b3)

    @pl.when(step == nsteps - 1)
    def _():
        v1o[...] = v1[...]
        v2o[...] = v2[...]
        v3o[...] = v3[...]
        i1o[...] = i1[...]
        i2o[...] = i2[...]
        i3o[...] = i3[...]


def _run_topk(q, k):
    nq = q.shape[0]
    grid = (k.shape[0] // _K_BLOCK,)
    return pl.pallas_call(
        _topk_body,
        grid=grid,
        in_specs=[
            pl.BlockSpec((nq, q.shape[1]), lambda i: (0, 0)),
            pl.BlockSpec((_K_BLOCK, k.shape[1]), lambda i: (i, 0)),
        ],
        out_specs=[pl.BlockSpec((nq, 1), lambda i: (0, 0))] * 6,
        out_shape=[jax.ShapeDtypeStruct((nq, 1), jnp.float32)] * 3
        + [jax.ShapeDtypeStruct((nq, 1), jnp.int32)] * 3,
        scratch_shapes=[pltpu.VMEM((nq, 1), jnp.float32)] * 4
        + [pltpu.VMEM((nq, 1), jnp.int32)] * 3,
    )(q, k)


def _sc_gather(table, idx):
    n_idx = idx.shape[1]
    mesh = plsc.VectorSubcoreMesh(core_axis_name="core",
                                  subcore_axis_name="subcore")

    @pl.kernel(out_type=jax.ShapeDtypeStruct((n_idx, table.shape[1]),
                                             table.dtype),
               mesh=mesh)
    def _gather_kernel(x_hbm, i_hbm, o_hbm):
        def body(i_vmem, o_vmem):
            pltpu.sync_copy(x_hbm.at[i_vmem.at[0]], o_vmem)

        pltpu.emit_pipeline(
            body,
            grid=(n_idx // _GATHER_WINDOW,),
            in_specs=[pl.BlockSpec((1, _GATHER_WINDOW),
                                   index_map=lambda i: (0, i))],
            out_specs=[pl.BlockSpec((_GATHER_WINDOW, table.shape[1]),
                                    index_map=lambda i: (i, 0))],
            core_axis_name="subcore",
            dimension_semantics=(pltpu.PARALLEL,),
        )(i_hbm, o_hbm)

    return _gather_kernel(table, idx)


def _finish_body(g_ref, w1, w2, w3, i1, i2, i3,
                 wfc_ref, bfc_ref, wom_ref, bom_ref, o_ref):
    n = o_ref.shape[0]
    d = o_ref.shape[1]

    def _half(g, idx):
        par = (idx[...] % 2) == 1
        return jnp.where(par, g[:, d:2 * d], g[:, 0:d])

    g0 = _half(g_ref[0:n], i1)
    g1 = _half(g_ref[n:2 * n], i2)
    g2 = _half(g_ref[2 * n:3 * n], i3)
    a, b, c = w1[...], w2[...], w3[...]
    agg = (g0 * a + g1 * b + g2 * c) / (a + b + c)
    t = jax.lax.dot_general(agg, wfc_ref[...], (((1,), (1,)), ((), ())),
                            preferred_element_type=jnp.float32) + bfc_ref[...]
    o_ref[...] = jax.lax.dot_general(t, wom_ref[...], (((1,), (1,)), ((), ())),
                                     preferred_element_type=jnp.float32) + bom_ref[...]


def _run_finish(gathered, v1, v2, v3, i1, i2, i3, W_fc, b_fc, W_om, b_om):
    nq = v1.shape[0]
    d = W_fc.shape[0]
    return pl.pallas_call(
        _finish_body,
        out_shape=jax.ShapeDtypeStruct((nq, d), jnp.float32),
    )(gathered, v1, v2, v3, i1, i2, i3,
      W_fc, b_fc.reshape(1, d), W_om, b_om.reshape(1, d))


def kernel(new_node_features, existing_node_features, W_fc, b_fc, W_om, b_om):
    v1, v2, v3, i1, i2, i3 = _run_topk(new_node_features,
                                       existing_node_features)
    nk, d = existing_node_features.shape
    table2 = existing_node_features.reshape(nk // 2, 2 * d)
    idx = jnp.concatenate([i1, i2, i3], axis=0).reshape(1, -1) // 2
    gathered = _sc_gather(table2, idx)
    return _run_finish(gathered, v1, v2, v3, i1, i2, i3,
                       W_fc, b_fc, W_om, b_om)


# speedup vs baseline: 2.2801x; 1.0280x over previous
import jax
import jax.numpy as jnp
from jax.experimental import pallas as pl
from jax.experimental.pallas import tpu as pltpu
from jax.experimental.pallas import tpu_sc as plsc

_K_BLOCK = 2000
_GATHER_WINDOW = 128
_NEG = -3.0e38
_HIGHEST = jax.lax.Precision.HIGHEST


def _topk_body(q_ref, k_ref, v1o, v2o, v3o, i1o, i2o, i3o,
               qn, v1, v2, v3, i1, i2, i3):
    step = pl.program_id(0)
    nsteps = pl.num_programs(0)

    @pl.when(step == 0)
    def _():
        q0 = q_ref[...]
        qn[...] = 1.0 / jnp.sqrt(jnp.sum(q0 * q0, axis=1, keepdims=True))
        v1[...] = jnp.full(v1.shape, _NEG, jnp.float32)
        v2[...] = jnp.full(v2.shape, _NEG, jnp.float32)
        v3[...] = jnp.full(v3.shape, _NEG, jnp.float32)
        i1[...] = jnp.zeros(i1.shape, jnp.int32)
        i2[...] = jnp.zeros(i2.shape, jnp.int32)
        i3[...] = jnp.zeros(i3.shape, jnp.int32)

    kb = k_ref[...]
    sq = kb * kb
    ones = jnp.ones((8, kb.shape[1]), jnp.float32)
    knsq = jax.lax.dot_general(ones, sq, (((1,), (1,)), ((), ())),
                               preferred_element_type=jnp.float32,
                               precision=_HIGHEST)[0:1]
    rkn = 1.0 / (jnp.sqrt(knsq) + 1e-30)
    s = jax.lax.dot_general(q_ref[...], kb, (((1,), (1,)), ((), ())),
                            preferred_element_type=jnp.float32)
    sim = s * rkn

    iota = jax.lax.broadcasted_iota(jnp.int32, sim.shape, 1) + step * _K_BLOCK
    big = jnp.int32(2 ** 30)

    m1 = jnp.max(sim, axis=1, keepdims=True)
    b1 = jnp.min(jnp.where(sim == m1, iota, big), axis=1, keepdims=True)
    sim_m = jnp.where(iota == b1, _NEG, sim)
    m2 = jnp.max(sim_m, axis=1, keepdims=True)
    b2 = jnp.min(jnp.where(sim_m == m2, iota, big), axis=1, keepdims=True)
    sim_m2 = jnp.where(iota == b2, _NEG, sim_m)
    m3 = jnp.max(sim_m2, axis=1, keepdims=True)
    b3 = jnp.min(jnp.where(sim_m2 == m3, iota, big), axis=1, keepdims=True)

    def _insert(cv, ci):
        pv1, pv2, pv3 = v1[...], v2[...], v3[...]
        pi1, pi2, pi3 = i1[...], i2[...], i3[...]
        g1 = cv > pv1
        g2 = cv > pv2
        g3 = cv > pv3
        v1[...] = jnp.where(g1, cv, pv1)
        i1[...] = jnp.where(g1, ci, pi1)
        v2[...] = jnp.where(g1, pv1, jnp.where(g2, cv, pv2))
        i2[...] = jnp.where(g1, pi1, jnp.where(g2, ci, pi2))
        v3[...] = jnp.where(g1 | g2, pv2, jnp.where(g3, cv, pv3))
        i3[...] = jnp.where(g1 | g2, pi2, jnp.where(g3, ci, pi3))

    _insert(m1, b1)
    _insert(m2, b2)
    _insert(m3, b3)

    @pl.when(step == nsteps - 1)
    def _():
        v1o[...] = v1[...] * qn[...]
        v2o[...] = v2[...] * qn[...]
        v3o[...] = v3[...] * qn[...]
        i1o[...] = i1[...]
        i2o[...] = i2[...]
        i3o[...] = i3[...]


def _run_topk(q, k):
    nq = q.shape[0]
    grid = (k.shape[0] // _K_BLOCK,)
    return pl.pallas_call(
        _topk_body,
        grid=grid,
        in_specs=[
            pl.BlockSpec((nq, q.shape[1]), lambda i: (0, 0)),
            pl.BlockSpec((_K_BLOCK, k.shape[1]), lambda i: (i, 0)),
        ],
        out_specs=[pl.BlockSpec((nq, 1), lambda i: (0, 0))] * 6,
        out_shape=[jax.ShapeDtypeStruct((nq, 1), jnp.float32)] * 3
        + [jax.ShapeDtypeStruct((nq, 1), jnp.int32)] * 3,
        scratch_shapes=[pltpu.VMEM((nq, 1), jnp.float32)] * 4
        + [pltpu.VMEM((nq, 1), jnp.int32)] * 3,
    )(q, k)


def _sc_gather(table, idx):
    n_idx = idx.shape[1]
    mesh = plsc.VectorSubcoreMesh(core_axis_name="core",
                                  subcore_axis_name="subcore")

    @pl.kernel(out_type=jax.ShapeDtypeStruct((n_idx, table.shape[1]),
                                             table.dtype),
               mesh=mesh)
    def _gather_kernel(x_hbm, i_hbm, o_hbm):
        def body(i_vmem, o_vmem):
            pltpu.sync_copy(x_hbm.at[i_vmem.at[0]], o_vmem)

        pltpu.emit_pipeline(
            body,
            grid=(n_idx // _GATHER_WINDOW,),
            in_specs=[pl.BlockSpec((1, _GATHER_WINDOW),
                                   index_map=lambda i: (0, i))],
            out_specs=[pl.BlockSpec((_GATHER_WINDOW, table.shape[1]),
                                    index_map=lambda i: (i, 0))],
            core_axis_name="subcore",
            dimension_semantics=(pltpu.PARALLEL,),
        )(i_hbm, o_hbm)

    return _gather_kernel(table, idx)


def _finish_body(g_ref, w1, w2, w3, i1, i2, i3,
                 wfc_ref, bfc_ref, wom_ref, bom_ref, o_ref):
    n = o_ref.shape[0]
    d = o_ref.shape[1]

    def _half(g, idx):
        par = (idx[...] % 2) == 1
        return jnp.where(par, g[:, d:2 * d], g[:, 0:d])

    g0 = _half(g_ref[0:n], i1)
    g1 = _half(g_ref[n:2 * n], i2)
    g2 = _half(g_ref[2 * n:3 * n], i3)
    a, b, c = w1[...], w2[...], w3[...]
    agg = (g0 * a + g1 * b + g2 * c) / (a + b + c)
    t = jax.lax.dot_general(agg, wfc_ref[...], (((1,), (1,)), ((), ())),
                            preferred_element_type=jnp.float32) + bfc_ref[...]
    o_ref[...] = jax.lax.dot_general(t, wom_ref[...], (((1,), (1,)), ((), ())),
                                     preferred_element_type=jnp.float32) + bom_ref[...]


def _run_finish(gathered, v1, v2, v3, i1, i2, i3, W_fc, b_fc, W_om, b_om):
    nq = v1.shape[0]
    d = W_fc.shape[0]
    return pl.pallas_call(
        _finish_body,
        out_shape=jax.ShapeDtypeStruct((nq, d), jnp.float32),
    )(gathered, v1, v2, v3, i1, i2, i3,
      W_fc, b_fc.reshape(1, d), W_om, b_om.reshape(1, d))


def kernel(new_node_features, existing_node_features, W_fc, b_fc, W_om, b_om):
    v1, v2, v3, i1, i2, i3 = _run_topk(new_node_features,
                                       existing_node_features)
    nk, d = existing_node_features.shape
    table2 = existing_node_features.reshape(nk // 2, 2 * d)
    idx = jnp.concatenate([i1, i2, i3], axis=0).reshape(1, -1) // 2
    gathered = _sc_gather(table2, idx)
    return _run_finish(gathered, v1, v2, v3, i1, i2, i3,
                       W_fc, b_fc, W_om, b_om)
